# baseline (speedup 1.0000x reference)
import jax
import jax.numpy as jnp
from jax import lax
from jax.experimental import pallas as pl
from jax.experimental.pallas import tpu as pltpu
from jax.experimental.pallas import tpu_sc as plsc

_NC = 2
_NS = 16
_NW = _NC * _NS
_CH = 128


def _body(tab_hbm, idx_hbm, out_hbm, idx_v, rows_v, sem):
    n_fields, n_chunks, _ = idx_v.shape
    bpw, d = rows_v.shape
    w = lax.axis_index("s") * _NC + lax.axis_index("c")
    pltpu.sync_copy(idx_hbm.at[w], idx_v)
    base = w * bpw

    def field_step(f, carry):
        cps = [
            pltpu.async_copy(
                tab_hbm.at[idx_v.at[f, c]],
                rows_v.at[pl.ds(c * _CH, _CH)],
                sem,
            )
            for c in range(n_chunks)
        ]
        for cp in cps:
            cp.wait()
        pltpu.sync_copy(
            rows_v, out_hbm.at[pl.ds(base, bpw), pl.ds(f * d, d)]
        )
        return carry

    lax.fori_loop(0, n_fields, field_step, 0)


def kernel(metas, tables):
    f, b = metas.shape
    v, d = tables.shape[1], tables.shape[2]
    bpw = b // _NW
    n_chunks = bpw // _CH

    idx = metas.astype(jnp.int32) + (jnp.arange(f, dtype=jnp.int32) * v)[:, None]
    idx = idx.reshape(f, _NW, n_chunks, _CH).transpose(1, 0, 2, 3)
    tab = tables.reshape(f * v, d)

    run = pl.kernel(
        _body,
        out_type=jax.ShapeDtypeStruct((b, f * d), jnp.float32),
        mesh=plsc.VectorSubcoreMesh(core_axis_name="c", subcore_axis_name="s"),
        scratch_types=[
            pltpu.VMEM((f, n_chunks, _CH), jnp.int32),
            pltpu.VMEM((bpw, d), jnp.float32),
            pltpu.SemaphoreType.DMA,
        ],
        compiler_params=pltpu.CompilerParams(use_tc_tiling_on_sc=False),
    )
    return run(tab, idx)
